# baseline (speedup 1.0000x reference)
import functools

import jax
import jax.numpy as jnp
import numpy as np
from jax import lax
from jax.experimental import pallas as pl
from jax.experimental.pallas import tpu as pltpu
from jax.experimental.pallas import tpu_sc as plsc

_B = 8
_N = 20000
_PRE = 6000
_MAXOUT = 1000
_THR = 0.7
_NSORT = 32768
_ROWS = 256
_TOPROWS = 48
_TOPK = _TOPROWS * 128


def _stage_params():
    rows = []
    kk = 2
    while kk <= _NSORT:
        j = kk // 2
        while j >= 1:
            dr = j // 128 if j >= 128 else 0
            jl = j if j < 128 else 0
            rows.append((j, kk, dr, jl))
            j //= 2
        kk *= 2
    return np.asarray(rows, dtype=np.int32)


_PARAMS = _stage_params()
_NSTAGES = _PARAMS.shape[0]


def _sort_body(params_ref, scores_ref, out_ref, key_ref, idx_ref, lin_ref):
    pid = pl.program_id(0)

    @pl.when(pid == 0)
    def _init():
        s = scores_ref[...]
        b = lax.bitcast_convert_type(s, jnp.int32)
        key = b ^ ((b >> 31) & jnp.int32(0x7FFFFFFF))
        r = lax.broadcasted_iota(jnp.int32, (_B, _ROWS, 128), 1)
        c = lax.broadcasted_iota(jnp.int32, (_B, _ROWS, 128), 2)
        lin = r * 128 + c
        lin_ref[...] = lin
        key_ref[:, 0:_ROWS, :] = key
        key_ref[:, _ROWS:, :] = key
        idx_ref[:, 0:_ROWS, :] = lin
        idx_ref[:, _ROWS:, :] = lin

    j = params_ref[pid, 0]
    kk = params_ref[pid, 1]
    dr = params_ref[pid, 2]
    jl = params_ref[pid, 3]
    sh_m = lax.rem(jnp.int32(128) - jl, jnp.int32(128))

    lin = lin_ref[...]
    xk = key_ref[:, 0:_ROWS, :]
    xi = idx_ref[:, 0:_ROWS, :]
    km = pltpu.roll(key_ref[:, pl.ds(dr, _ROWS), :], sh_m, axis=2)
    kp = pltpu.roll(key_ref[:, pl.ds(_ROWS - dr, _ROWS), :], jl, axis=2)
    im = pltpu.roll(idx_ref[:, pl.ds(dr, _ROWS), :], sh_m, axis=2)
    ip = pltpu.roll(idx_ref[:, pl.ds(_ROWS - dr, _ROWS), :], jl, axis=2)

    is_hi = (lin & j) != 0
    is_asc = (lin & kk) != 0
    pk = jnp.where(is_hi, kp, km)
    pi = jnp.where(is_hi, ip, im)
    self_wins = (xk > pk) | ((xk == pk) & (xi < pi))
    keep = self_wins ^ is_hi ^ is_asc
    nk = jnp.where(keep, xk, pk)
    ni = jnp.where(keep, xi, pi)
    key_ref[:, 0:_ROWS, :] = nk
    key_ref[:, _ROWS:, :] = nk
    idx_ref[:, 0:_ROWS, :] = ni
    idx_ref[:, _ROWS:, :] = ni

    @pl.when(pid == _NSTAGES - 1)
    def _fin():
        ii = idx_ref[:, 0:_TOPROWS, :]
        bidx = lax.broadcasted_iota(jnp.int32, (_B, _TOPROWS, 128), 0)
        out_ref[...] = ii + bidx * _N


def _topk_indices(scores_pad):
    return pl.pallas_call(
        _sort_body,
        grid_spec=pltpu.PrefetchScalarGridSpec(
            num_scalar_prefetch=1,
            grid=(_NSTAGES,),
            in_specs=[
                pl.BlockSpec((_B, _ROWS, 128), lambda i, p: (0, 0, 0)),
            ],
            out_specs=pl.BlockSpec((_B, _TOPROWS, 128), lambda i, p: (0, 0, 0)),
            scratch_shapes=[
                pltpu.VMEM((_B, 2 * _ROWS, 128), jnp.int32),
                pltpu.VMEM((_B, 2 * _ROWS, 128), jnp.int32),
                pltpu.VMEM((_B, _ROWS, 128), jnp.int32),
            ],
        ),
        out_shape=jax.ShapeDtypeStruct((_B, _TOPROWS, 128), jnp.int32),
    )(jnp.asarray(_PARAMS), scores_pad)


_NTILES = 32
_PER_TILE = (_B * _TOPK) // _NTILES
_CHUNKS = _PER_TILE // 128


def _sc_gather(idx3d, wide_tab):
    mesh = plsc.VectorSubcoreMesh(core_axis_name="c", subcore_axis_name="s")

    @functools.partial(
        pl.kernel,
        mesh=mesh,
        out_type=jax.ShapeDtypeStruct((_B * _TOPK, 128), jnp.float32),
        scratch_types=[
            pltpu.VMEM((_CHUNKS, 128), jnp.int32),
            pltpu.VMEM((128, 128), jnp.float32),
            pltpu.VMEM((128, 128), jnp.float32),
            pltpu.SemaphoreType.DMA,
            pltpu.SemaphoreType.DMA,
        ],
    )
    def gather_k(idx_hbm, tab_hbm, out, idx_v, buf0, buf1, sem0, sem1):
        wid = lax.axis_index("s") * 2 + lax.axis_index("c")
        pltpu.sync_copy(idx_hbm.at[wid], idx_v)
        bufs = (buf0, buf1)
        sems = (sem0, sem1)
        cps = [None] * _CHUNKS
        cps[0] = pltpu.async_copy(tab_hbm.at[idx_v.at[0]], bufs[0], sems[0])
        for c in range(_CHUNKS):
            if c + 1 < _CHUNKS:
                cps[c + 1] = pltpu.async_copy(tab_hbm.at[idx_v.at[c + 1]],
                                              bufs[(c + 1) % 2],
                                              sems[(c + 1) % 2])
            cps[c].wait()
            pltpu.sync_copy(bufs[c % 2],
                            out.at[pl.ds(wid * _PER_TILE + c * 128, 128), :])

    return gather_k(idx3d, wide_tab)


_BIG = 1 << 20


def _nms_body(ay1, ax1, ay2, ax2, dy, dx, dh, dw,
              oy1, ox1, oy2, ox2,
              y1r, x1r, y2r, x2r, arear, remr, cntr):
    h = ay2[...] - ay1[...]
    w = ax2[...] - ax1[...]
    cy = ay1[...] + 0.5 * h
    cx = ax1[...] + 0.5 * w
    cy = cy + (dy[...] * 0.1) * h
    cx = cx + (dx[...] * 0.1) * w
    h = h * jnp.exp(dh[...] * 0.2)
    w = w * jnp.exp(dw[...] * 0.2)
    y1 = cy - 0.5 * h
    x1 = cx - 0.5 * w
    y2 = y1 + h
    x2 = x1 + w
    y1 = jnp.clip(y1, 0.0, 1.0)
    x1 = jnp.clip(x1, 0.0, 1.0)
    y2 = jnp.clip(y2, 0.0, 1.0)
    x2 = jnp.clip(x2, 0.0, 1.0)
    y1r[...] = y1
    x1r[...] = x1
    y2r[...] = y2
    x2r[...] = x2
    arear[...] = jnp.maximum(y2 - y1, 0.0) * jnp.maximum(x2 - x1, 0.0)

    lin = lax.broadcasted_iota(jnp.int32, (_B, _TOPK), 1)
    remr[...] = (lin < _PRE).astype(jnp.int32)
    cntr[...] = jnp.zeros((_B, 128), jnp.int32)
    oy1[...] = jnp.zeros((_B, 1024), jnp.float32)
    ox1[...] = jnp.zeros((_B, 1024), jnp.float32)
    oy2[...] = jnp.zeros((_B, 1024), jnp.float32)
    ox2[...] = jnp.zeros((_B, 1024), jnp.float32)
    lane_o = lax.broadcasted_iota(jnp.int32, (_B, 1024), 1)

    def body(carry):
        rem = remr[...]
        cnt = cntr[...]
        cand = jnp.min(jnp.where(rem > 0, lin, _BIG), axis=1, keepdims=True)
        active = (cand < _BIG) & (cnt[:, 0:1] < _MAXOUT)
        active_b = jnp.broadcast_to(active, (_B, _TOPK))
        pick = (lin == cand) & active_b
        y1a = y1r[...]
        x1a = x1r[...]
        y2a = y2r[...]
        x2a = x2r[...]
        by1 = jnp.max(jnp.where(pick, y1a, -1.0), axis=1, keepdims=True)
        bx1 = jnp.max(jnp.where(pick, x1a, -1.0), axis=1, keepdims=True)
        by2 = jnp.max(jnp.where(pick, y2a, -1.0), axis=1, keepdims=True)
        bx2 = jnp.max(jnp.where(pick, x2a, -1.0), axis=1, keepdims=True)
        barea = (jnp.maximum(by2 - by1, 0.0) * jnp.maximum(bx2 - bx1, 0.0))
        yy1 = jnp.maximum(by1, y1a)
        xx1 = jnp.maximum(bx1, x1a)
        yy2 = jnp.minimum(by2, y2a)
        xx2 = jnp.minimum(bx2, x2a)
        inter = jnp.maximum(yy2 - yy1, 0.0) * jnp.maximum(xx2 - xx1, 0.0)
        iou = inter / (barea + arear[...] - inter + 1e-8)
        supp = (iou >= _THR) & active_b
        remr[...] = rem * (1 - (supp | pick).astype(jnp.int32))
        osel = (lane_o == cnt[:, 0:1]) & jnp.broadcast_to(active, (_B, 1024))
        oy1[...] = jnp.where(osel, jnp.broadcast_to(by1, (_B, 1024)), oy1[...])
        ox1[...] = jnp.where(osel, jnp.broadcast_to(bx1, (_B, 1024)), ox1[...])
        oy2[...] = jnp.where(osel, jnp.broadcast_to(by2, (_B, 1024)), oy2[...])
        ox2[...] = jnp.where(osel, jnp.broadcast_to(bx2, (_B, 1024)), ox2[...])
        cntr[...] = cnt + jnp.broadcast_to(active, (_B, 128)).astype(jnp.int32)
        return jnp.sum(active.astype(jnp.int32))

    lax.while_loop(lambda c: c > 0, body, jnp.int32(1))


def _decode_nms(planes):
    outs = pl.pallas_call(
        _nms_body,
        out_shape=[jax.ShapeDtypeStruct((_B, 1024), jnp.float32)] * 4,
        scratch_shapes=[pltpu.VMEM((_B, _TOPK), jnp.float32)] * 5
        + [pltpu.VMEM((_B, _TOPK), jnp.int32),
           pltpu.VMEM((_B, 128), jnp.int32)],
    )(*planes)
    return outs


def kernel(class_probs, bbox_offset, anchors):
    scores = class_probs[:, :, 1]
    spad = jnp.pad(scores, ((0, 0), (0, _NSORT - _N)),
                   constant_values=-jnp.inf).reshape(_B, _ROWS, 128)
    idx3 = _topk_indices(spad)
    idx2 = idx3.reshape(_NTILES, _CHUNKS, 128)
    big = jnp.concatenate([anchors.reshape(_B * _N, 4),
                           bbox_offset.reshape(_B * _N, 4)], axis=1)
    wide = jnp.pad(big, ((0, 0), (0, 120)))
    g = _sc_gather(idx2, wide)
    planes = tuple(g[:, c].reshape(_B, _TOPK) for c in range(8))
    oy1, ox1, oy2, ox2 = _decode_nms(planes)
    props = jnp.stack([oy1, ox1, oy2, ox2], axis=-1)[:, :_MAXOUT, :]
    return props


# speedup vs baseline: 1.2109x; 1.2109x over previous
import functools

import jax
import jax.numpy as jnp
import numpy as np
from jax import lax
from jax.experimental import pallas as pl
from jax.experimental.pallas import tpu as pltpu
from jax.experimental.pallas import tpu_sc as plsc

_B = 8
_N = 20000
_PRE = 6000
_MAXOUT = 1000
_THR = 0.7
_NSORT = 32768
_ROWS = 256
_TOPROWS = 48
_TOPK = _TOPROWS * 128


def _stage_params():
    rows = []
    kk = 2
    while kk <= _NSORT:
        j = kk // 2
        while j >= 1:
            dr = j // 128 if j >= 128 else 0
            jl = j if j < 128 else 0
            rows.append((j, kk, dr, jl))
            j //= 2
        kk *= 2
    return np.asarray(rows, dtype=np.int32)


_PARAMS = _stage_params()
_NSTAGES = _PARAMS.shape[0]


def _sort_body(params_ref, scores_ref, out_ref, key_ref, idx_ref, lin_ref):
    pid = pl.program_id(0)

    @pl.when(pid == 0)
    def _init():
        s = scores_ref[...]
        b = lax.bitcast_convert_type(s, jnp.int32)
        key = b ^ ((b >> 31) & jnp.int32(0x7FFFFFFF))
        r = lax.broadcasted_iota(jnp.int32, (_B, _ROWS, 128), 1)
        c = lax.broadcasted_iota(jnp.int32, (_B, _ROWS, 128), 2)
        lin = r * 128 + c
        lin_ref[...] = lin
        key_ref[:, 0:_ROWS, :] = key
        key_ref[:, _ROWS:, :] = key
        idx_ref[:, 0:_ROWS, :] = lin
        idx_ref[:, _ROWS:, :] = lin

    j = params_ref[pid, 0]
    kk = params_ref[pid, 1]
    dr = params_ref[pid, 2]
    jl = params_ref[pid, 3]
    sh_m = lax.rem(jnp.int32(128) - jl, jnp.int32(128))

    lin = lin_ref[...]
    xk = key_ref[:, 0:_ROWS, :]
    xi = idx_ref[:, 0:_ROWS, :]
    km = pltpu.roll(key_ref[:, pl.ds(dr, _ROWS), :], sh_m, axis=2)
    kp = pltpu.roll(key_ref[:, pl.ds(_ROWS - dr, _ROWS), :], jl, axis=2)
    im = pltpu.roll(idx_ref[:, pl.ds(dr, _ROWS), :], sh_m, axis=2)
    ip = pltpu.roll(idx_ref[:, pl.ds(_ROWS - dr, _ROWS), :], jl, axis=2)

    is_hi = (lin & j) != 0
    is_asc = (lin & kk) != 0
    pk = jnp.where(is_hi, kp, km)
    pi = jnp.where(is_hi, ip, im)
    self_wins = (xk > pk) | ((xk == pk) & (xi < pi))
    keep = self_wins ^ is_hi ^ is_asc
    nk = jnp.where(keep, xk, pk)
    ni = jnp.where(keep, xi, pi)
    key_ref[:, 0:_ROWS, :] = nk
    key_ref[:, _ROWS:, :] = nk
    idx_ref[:, 0:_ROWS, :] = ni
    idx_ref[:, _ROWS:, :] = ni

    @pl.when(pid == _NSTAGES - 1)
    def _fin():
        ii = idx_ref[:, 0:_TOPROWS, :]
        bidx = lax.broadcasted_iota(jnp.int32, (_B, _TOPROWS, 128), 0)
        out_ref[...] = ii + bidx * _N


def _topk_indices(scores_pad):
    return pl.pallas_call(
        _sort_body,
        grid_spec=pltpu.PrefetchScalarGridSpec(
            num_scalar_prefetch=1,
            grid=(_NSTAGES,),
            in_specs=[
                pl.BlockSpec((_B, _ROWS, 128), lambda i, p: (0, 0, 0)),
            ],
            out_specs=pl.BlockSpec((_B, _TOPROWS, 128), lambda i, p: (0, 0, 0)),
            scratch_shapes=[
                pltpu.VMEM((_B, 2 * _ROWS, 128), jnp.int32),
                pltpu.VMEM((_B, 2 * _ROWS, 128), jnp.int32),
                pltpu.VMEM((_B, _ROWS, 128), jnp.int32),
            ],
        ),
        out_shape=jax.ShapeDtypeStruct((_B, _TOPROWS, 128), jnp.int32),
    )(jnp.asarray(_PARAMS), scores_pad)


_NTILES = 32
_PER_TILE = (_B * _TOPK) // _NTILES
_CHUNKS = _PER_TILE // 128


def _sc_gather(idx3d, wide_tab):
    mesh = plsc.VectorSubcoreMesh(core_axis_name="c", subcore_axis_name="s")

    @functools.partial(
        pl.kernel,
        mesh=mesh,
        out_type=jax.ShapeDtypeStruct((_B * _TOPK, 128), jnp.float32),
        scratch_types=[
            pltpu.VMEM((_CHUNKS, 128), jnp.int32),
            pltpu.VMEM((128, 128), jnp.float32),
            pltpu.VMEM((128, 128), jnp.float32),
            pltpu.SemaphoreType.DMA,
            pltpu.SemaphoreType.DMA,
        ],
    )
    def gather_k(idx_hbm, tab_hbm, out, idx_v, buf0, buf1, sem0, sem1):
        wid = lax.axis_index("s") * 2 + lax.axis_index("c")
        pltpu.sync_copy(idx_hbm.at[wid], idx_v)
        bufs = (buf0, buf1)
        sems = (sem0, sem1)
        cps = [None] * _CHUNKS
        cps[0] = pltpu.async_copy(tab_hbm.at[idx_v.at[0]], bufs[0], sems[0])
        for c in range(_CHUNKS):
            if c + 1 < _CHUNKS:
                cps[c + 1] = pltpu.async_copy(tab_hbm.at[idx_v.at[c + 1]],
                                              bufs[(c + 1) % 2],
                                              sems[(c + 1) % 2])
            cps[c].wait()
            pltpu.sync_copy(bufs[c % 2],
                            out.at[pl.ds(wid * _PER_TILE + c * 128, 128), :])

    return gather_k(idx3d, wide_tab)


_BIG = 1 << 20
_W = 2048


def _nms_body(ay1, ax1, ay2, ax2, dy, dx, dh, dw,
              oy1, ox1, oy2, ox2,
              y1r, x1r, y2r, x2r, arear, remr, cntr):
    h = ay2[...] - ay1[...]
    w = ax2[...] - ax1[...]
    cy = ay1[...] + 0.5 * h
    cx = ax1[...] + 0.5 * w
    cy = cy + (dy[...] * 0.1) * h
    cx = cx + (dx[...] * 0.1) * w
    h = h * jnp.exp(dh[...] * 0.2)
    w = w * jnp.exp(dw[...] * 0.2)
    y1 = cy - 0.5 * h
    x1 = cx - 0.5 * w
    y2 = y1 + h
    x2 = x1 + w
    y1 = jnp.clip(y1, 0.0, 1.0)
    x1 = jnp.clip(x1, 0.0, 1.0)
    y2 = jnp.clip(y2, 0.0, 1.0)
    x2 = jnp.clip(x2, 0.0, 1.0)
    y1r[...] = y1
    x1r[...] = x1
    y2r[...] = y2
    x2r[...] = x2
    arear[...] = jnp.maximum(y2 - y1, 0.0) * jnp.maximum(x2 - x1, 0.0)

    cntr[...] = jnp.zeros((_B, 128), jnp.int32)
    oy1[...] = jnp.zeros((_B, 1024), jnp.float32)
    ox1[...] = jnp.zeros((_B, 1024), jnp.float32)
    oy2[...] = jnp.zeros((_B, 1024), jnp.float32)
    ox2[...] = jnp.zeros((_B, 1024), jnp.float32)
    lane_o = lax.broadcasted_iota(jnp.int32, (_B, 1024), 1)
    lin_w = lax.broadcasted_iota(jnp.int32, (_B, _W), 1)

    for w in range(_TOPK // _W):
        sl = pl.ds(w * _W, _W)
        nvalid = _PRE - w * _W
        cnt0 = cntr[...][:, 0:1]

        @pl.when(jnp.min(cnt0) < _MAXOUT)
        def _window():
            y1w = y1r[:, sl]
            x1w = x1r[:, sl]
            y2w = y2r[:, sl]
            x2w = x2r[:, sl]
            aw = arear[:, sl]
            valid = (lin_w < nvalid)

            if w == 0:
                remr[...] = valid.astype(jnp.int32)
            else:
                def ext(j, supp):
                    oh = lane_o == j
                    sy1 = jnp.max(jnp.where(oh, oy1[...], -1.0), axis=1,
                                  keepdims=True)
                    sx1 = jnp.max(jnp.where(oh, ox1[...], -1.0), axis=1,
                                  keepdims=True)
                    sy2 = jnp.max(jnp.where(oh, oy2[...], -1.0), axis=1,
                                  keepdims=True)
                    sx2 = jnp.max(jnp.where(oh, ox2[...], -1.0), axis=1,
                                  keepdims=True)
                    sarea = (jnp.maximum(sy2 - sy1, 0.0)
                             * jnp.maximum(sx2 - sx1, 0.0))
                    yy1 = jnp.maximum(sy1, y1w)
                    xx1 = jnp.maximum(sx1, x1w)
                    yy2 = jnp.minimum(sy2, y2w)
                    xx2 = jnp.minimum(sx2, x2w)
                    inter = (jnp.maximum(yy2 - yy1, 0.0)
                             * jnp.maximum(xx2 - xx1, 0.0))
                    iou = inter / (sarea + aw - inter + 1e-8)
                    return supp | (iou >= _THR).astype(jnp.int32)

                maxc = jnp.max(cntr[...][:, 0:1])
                supp0 = lax.fori_loop(0, maxc, ext,
                                      jnp.zeros((_B, _W), jnp.int32))
                remr[...] = (valid & (supp0 == 0)).astype(jnp.int32)

            def body(carry):
                rem = remr[...]
                cnt = cntr[...]
                cand = jnp.min(jnp.where(rem > 0, lin_w, _BIG), axis=1,
                               keepdims=True)
                active = (cand < _BIG) & (cnt[:, 0:1] < _MAXOUT)
                active_b = jnp.broadcast_to(active, (_B, _W))
                pick = (lin_w == cand) & active_b
                by1 = jnp.max(jnp.where(pick, y1w, -1.0), axis=1,
                              keepdims=True)
                bx1 = jnp.max(jnp.where(pick, x1w, -1.0), axis=1,
                              keepdims=True)
                by2 = jnp.max(jnp.where(pick, y2w, -1.0), axis=1,
                              keepdims=True)
                bx2 = jnp.max(jnp.where(pick, x2w, -1.0), axis=1,
                              keepdims=True)
                barea = (jnp.maximum(by2 - by1, 0.0)
                         * jnp.maximum(bx2 - bx1, 0.0))
                yy1 = jnp.maximum(by1, y1w)
                xx1 = jnp.maximum(bx1, x1w)
                yy2 = jnp.minimum(by2, y2w)
                xx2 = jnp.minimum(bx2, x2w)
                inter = (jnp.maximum(yy2 - yy1, 0.0)
                         * jnp.maximum(xx2 - xx1, 0.0))
                iou = inter / (barea + aw - inter + 1e-8)
                supp = (iou >= _THR) & active_b
                remr[...] = rem * (1 - (supp | pick).astype(jnp.int32))
                osel = ((lane_o == cnt[:, 0:1])
                        & jnp.broadcast_to(active, (_B, 1024)))
                oy1[...] = jnp.where(osel,
                                     jnp.broadcast_to(by1, (_B, 1024)),
                                     oy1[...])
                ox1[...] = jnp.where(osel,
                                     jnp.broadcast_to(bx1, (_B, 1024)),
                                     ox1[...])
                oy2[...] = jnp.where(osel,
                                     jnp.broadcast_to(by2, (_B, 1024)),
                                     oy2[...])
                ox2[...] = jnp.where(osel,
                                     jnp.broadcast_to(bx2, (_B, 1024)),
                                     ox2[...])
                cntr[...] = cnt + jnp.broadcast_to(active,
                                                   (_B, 128)).astype(jnp.int32)
                return jnp.sum(active.astype(jnp.int32))

            lax.while_loop(lambda c: c > 0, body, jnp.int32(1))


def _decode_nms(planes):
    outs = pl.pallas_call(
        _nms_body,
        out_shape=[jax.ShapeDtypeStruct((_B, 1024), jnp.float32)] * 4,
        scratch_shapes=[pltpu.VMEM((_B, _TOPK), jnp.float32)] * 5
        + [pltpu.VMEM((_B, _W), jnp.int32),
           pltpu.VMEM((_B, 128), jnp.int32)],
    )(*planes)
    return outs


def kernel(class_probs, bbox_offset, anchors):
    scores = class_probs[:, :, 1]
    spad = jnp.pad(scores, ((0, 0), (0, _NSORT - _N)),
                   constant_values=-jnp.inf).reshape(_B, _ROWS, 128)
    idx3 = _topk_indices(spad)
    idx2 = idx3.reshape(_NTILES, _CHUNKS, 128)
    big = jnp.concatenate([anchors.reshape(_B * _N, 4),
                           bbox_offset.reshape(_B * _N, 4)], axis=1)
    wide = jnp.pad(big, ((0, 0), (0, 120)))
    g = _sc_gather(idx2, wide)
    planes = tuple(g[:, c].reshape(_B, _TOPK) for c in range(8))
    oy1, ox1, oy2, ox2 = _decode_nms(planes)
    props = jnp.stack([oy1, ox1, oy2, ox2], axis=-1)[:, :_MAXOUT, :]
    return props


# speedup vs baseline: 1.2508x; 1.0330x over previous
import functools

import jax
import jax.numpy as jnp
import numpy as np
from jax import lax
from jax.experimental import pallas as pl
from jax.experimental.pallas import tpu as pltpu
from jax.experimental.pallas import tpu_sc as plsc

_B = 8
_N = 20000
_PRE = 6000
_MAXOUT = 1000
_THR = 0.7
_NSORT = 32768
_ROWS = 256
_TOPROWS = 48
_TOPK = _TOPROWS * 128


def _stage_params():
    rows = []
    kk = 2
    while kk <= _NSORT:
        j = kk // 2
        while j >= 1:
            dr = j // 128 if j >= 128 else 0
            jl = j if j < 128 else 0
            rows.append((j, kk, dr, jl))
            j //= 2
        kk *= 2
    return np.asarray(rows, dtype=np.int32)


_PARAMS = _stage_params()
_NSTAGES = _PARAMS.shape[0]


def _sort_body(params_ref, scores_ref, out_ref, key_ref, idx_ref, lin_ref):
    pid = pl.program_id(0)

    @pl.when(pid == 0)
    def _init():
        s = scores_ref[...]
        b = lax.bitcast_convert_type(s, jnp.int32)
        key = b ^ ((b >> 31) & jnp.int32(0x7FFFFFFF))
        r = lax.broadcasted_iota(jnp.int32, (_B, _ROWS, 128), 1)
        c = lax.broadcasted_iota(jnp.int32, (_B, _ROWS, 128), 2)
        lin = r * 128 + c
        lin_ref[...] = lin
        key_ref[:, 0:_ROWS, :] = key
        key_ref[:, _ROWS:, :] = key
        idx_ref[:, 0:_ROWS, :] = lin
        idx_ref[:, _ROWS:, :] = lin

    j = params_ref[pid, 0]
    kk = params_ref[pid, 1]
    dr = params_ref[pid, 2]
    jl = params_ref[pid, 3]
    sh_m = lax.rem(jnp.int32(128) - jl, jnp.int32(128))

    lin = lin_ref[...]
    xk = key_ref[:, 0:_ROWS, :]
    xi = idx_ref[:, 0:_ROWS, :]
    km = pltpu.roll(key_ref[:, pl.ds(dr, _ROWS), :], sh_m, axis=2)
    kp = pltpu.roll(key_ref[:, pl.ds(_ROWS - dr, _ROWS), :], jl, axis=2)
    im = pltpu.roll(idx_ref[:, pl.ds(dr, _ROWS), :], sh_m, axis=2)
    ip = pltpu.roll(idx_ref[:, pl.ds(_ROWS - dr, _ROWS), :], jl, axis=2)

    is_hi = (lin & j) != 0
    is_asc = (lin & kk) != 0
    pk = jnp.where(is_hi, kp, km)
    pi = jnp.where(is_hi, ip, im)
    self_wins = (xk > pk) | ((xk == pk) & (xi < pi))
    keep = self_wins ^ is_hi ^ is_asc
    nk = jnp.where(keep, xk, pk)
    ni = jnp.where(keep, xi, pi)
    key_ref[:, 0:_ROWS, :] = nk
    key_ref[:, _ROWS:, :] = nk
    idx_ref[:, 0:_ROWS, :] = ni
    idx_ref[:, _ROWS:, :] = ni

    @pl.when(pid == _NSTAGES - 1)
    def _fin():
        ii = idx_ref[:, 0:_TOPROWS, :]
        bidx = lax.broadcasted_iota(jnp.int32, (_B, _TOPROWS, 128), 0)
        out_ref[...] = ii + bidx * _N


def _topk_indices(scores_pad):
    return pl.pallas_call(
        _sort_body,
        grid_spec=pltpu.PrefetchScalarGridSpec(
            num_scalar_prefetch=1,
            grid=(_NSTAGES,),
            in_specs=[
                pl.BlockSpec((_B, _ROWS, 128), lambda i, p: (0, 0, 0)),
            ],
            out_specs=pl.BlockSpec((_B, _TOPROWS, 128), lambda i, p: (0, 0, 0)),
            scratch_shapes=[
                pltpu.VMEM((_B, 2 * _ROWS, 128), jnp.int32),
                pltpu.VMEM((_B, 2 * _ROWS, 128), jnp.int32),
                pltpu.VMEM((_B, _ROWS, 128), jnp.int32),
            ],
        ),
        out_shape=jax.ShapeDtypeStruct((_B, _TOPROWS, 128), jnp.int32),
    )(jnp.asarray(_PARAMS), scores_pad)


_NTILES = 32
_PER_TILE = (_B * _TOPK) // _NTILES
_CHUNKS = _PER_TILE // 128


def _sc_gather(idx3d, wide_tab):
    mesh = plsc.VectorSubcoreMesh(core_axis_name="c", subcore_axis_name="s")

    @functools.partial(
        pl.kernel,
        mesh=mesh,
        out_type=jax.ShapeDtypeStruct((_B * _TOPK, 128), jnp.float32),
        scratch_types=[
            pltpu.VMEM((_CHUNKS, 128), jnp.int32),
            pltpu.VMEM((128, 128), jnp.float32),
            pltpu.VMEM((128, 128), jnp.float32),
            pltpu.SemaphoreType.DMA,
            pltpu.SemaphoreType.DMA,
        ],
    )
    def gather_k(idx_hbm, tab_hbm, out, idx_v, buf0, buf1, sem0, sem1):
        wid = lax.axis_index("s") * 2 + lax.axis_index("c")
        pltpu.sync_copy(idx_hbm.at[wid], idx_v)
        bufs = (buf0, buf1)
        sems = (sem0, sem1)
        cps = [None] * _CHUNKS
        cps[0] = pltpu.async_copy(tab_hbm.at[idx_v.at[0]], bufs[0], sems[0])
        for c in range(_CHUNKS):
            if c + 1 < _CHUNKS:
                cps[c + 1] = pltpu.async_copy(tab_hbm.at[idx_v.at[c + 1]],
                                              bufs[(c + 1) % 2],
                                              sems[(c + 1) % 2])
            cps[c].wait()
            pltpu.sync_copy(bufs[c % 2],
                            out.at[pl.ds(wid * _PER_TILE + c * 128, 128), :])

    return gather_k(idx3d, wide_tab)


_BIG = 1 << 20
_W = 2048
_SPB = 4


def _nms_body(ay1, ax1, ay2, ax2, dy, dx, dh, dw,
              oy1, ox1, oy2, ox2,
              y1r, x1r, y2r, x2r, arear, remr, cntr):
    h = ay2[...] - ay1[...]
    w = ax2[...] - ax1[...]
    cy = ay1[...] + 0.5 * h
    cx = ax1[...] + 0.5 * w
    cy = cy + (dy[...] * 0.1) * h
    cx = cx + (dx[...] * 0.1) * w
    h = h * jnp.exp(dh[...] * 0.2)
    w = w * jnp.exp(dw[...] * 0.2)
    y1 = cy - 0.5 * h
    x1 = cx - 0.5 * w
    y2 = y1 + h
    x2 = x1 + w
    y1 = jnp.clip(y1, 0.0, 1.0)
    x1 = jnp.clip(x1, 0.0, 1.0)
    y2 = jnp.clip(y2, 0.0, 1.0)
    x2 = jnp.clip(x2, 0.0, 1.0)
    y1r[...] = y1
    x1r[...] = x1
    y2r[...] = y2
    x2r[...] = x2
    arear[...] = jnp.maximum(y2 - y1, 0.0) * jnp.maximum(x2 - x1, 0.0)

    cntr[...] = jnp.zeros((_B, 128), jnp.int32)
    oy1[...] = jnp.zeros((_B, 1024), jnp.float32)
    ox1[...] = jnp.zeros((_B, 1024), jnp.float32)
    oy2[...] = jnp.zeros((_B, 1024), jnp.float32)
    ox2[...] = jnp.zeros((_B, 1024), jnp.float32)
    lane_o = lax.broadcasted_iota(jnp.int32, (_B, 1024), 1)
    lin_w = lax.broadcasted_iota(jnp.int32, (_B, _W), 1)

    for w in range(_TOPK // _W):
        sl = pl.ds(w * _W, _W)
        nvalid = _PRE - w * _W
        cnt0 = cntr[...][:, 0:1]

        @pl.when(jnp.min(cnt0) < _MAXOUT)
        def _window():
            y1w = y1r[:, sl]
            x1w = x1r[:, sl]
            y2w = y2r[:, sl]
            x2w = x2r[:, sl]
            aw = arear[:, sl]
            valid = (lin_w < nvalid)

            if w == 0:
                remr[...] = valid.astype(jnp.int32)
            else:
                def ext(j, supp):
                    oh = lane_o == j
                    sy1 = jnp.max(jnp.where(oh, oy1[...], -1.0), axis=1,
                                  keepdims=True)
                    sx1 = jnp.max(jnp.where(oh, ox1[...], -1.0), axis=1,
                                  keepdims=True)
                    sy2 = jnp.max(jnp.where(oh, oy2[...], -1.0), axis=1,
                                  keepdims=True)
                    sx2 = jnp.max(jnp.where(oh, ox2[...], -1.0), axis=1,
                                  keepdims=True)
                    sarea = (jnp.maximum(sy2 - sy1, 0.0)
                             * jnp.maximum(sx2 - sx1, 0.0))
                    yy1 = jnp.maximum(sy1, y1w)
                    xx1 = jnp.maximum(sx1, x1w)
                    yy2 = jnp.minimum(sy2, y2w)
                    xx2 = jnp.minimum(sx2, x2w)
                    inter = (jnp.maximum(yy2 - yy1, 0.0)
                             * jnp.maximum(xx2 - xx1, 0.0))
                    iou = inter / (sarea + aw - inter + 1e-8)
                    return supp | (iou >= _THR).astype(jnp.int32)

                maxc = jnp.max(cntr[...][:, 0:1])
                supp0 = lax.fori_loop(0, maxc, ext,
                                      jnp.zeros((_B, _W), jnp.int32))
                remr[...] = (valid & (supp0 == 0)).astype(jnp.int32)

            def body(carry):
                rem = remr[...]
                cnt1 = cntr[...][:, 0:1]
                o1 = oy1[...]
                o2 = ox1[...]
                o3 = oy2[...]
                o4 = ox2[...]
                nact = jnp.int32(0)
                for _ in range(_SPB):
                    cand = jnp.min(jnp.where(rem > 0, lin_w, _BIG), axis=1,
                                   keepdims=True)
                    cand = jnp.where(cnt1 < _MAXOUT, cand, _BIG)
                    active = cand < _BIG
                    pick = lin_w == cand
                    by1 = jnp.max(jnp.where(pick, y1w, -1.0), axis=1,
                                  keepdims=True)
                    bx1 = jnp.max(jnp.where(pick, x1w, -1.0), axis=1,
                                  keepdims=True)
                    by2 = jnp.max(jnp.where(pick, y2w, -1.0), axis=1,
                                  keepdims=True)
                    bx2 = jnp.max(jnp.where(pick, x2w, -1.0), axis=1,
                                  keepdims=True)
                    barea = (jnp.maximum(by2 - by1, 0.0)
                             * jnp.maximum(bx2 - bx1, 0.0))
                    yy1 = jnp.maximum(by1, y1w)
                    xx1 = jnp.maximum(bx1, x1w)
                    yy2 = jnp.minimum(by2, y2w)
                    xx2 = jnp.minimum(bx2, x2w)
                    inter = (jnp.maximum(yy2 - yy1, 0.0)
                             * jnp.maximum(xx2 - xx1, 0.0))
                    iou = inter / (barea + aw - inter + 1e-8)
                    supp = iou >= _THR
                    rem = rem * (1 - (supp | pick).astype(jnp.int32))
                    wsel = jnp.where(active, cnt1, 2000)
                    osel = lane_o == wsel
                    o1 = jnp.where(osel, jnp.broadcast_to(by1, (_B, 1024)),
                                   o1)
                    o2 = jnp.where(osel, jnp.broadcast_to(bx1, (_B, 1024)),
                                   o2)
                    o3 = jnp.where(osel, jnp.broadcast_to(by2, (_B, 1024)),
                                   o3)
                    o4 = jnp.where(osel, jnp.broadcast_to(bx2, (_B, 1024)),
                                   o4)
                    cnt1 = cnt1 + active.astype(jnp.int32)
                    nact = jnp.sum(active.astype(jnp.int32))
                remr[...] = rem
                cntr[...] = jnp.broadcast_to(cnt1, (_B, 128))
                oy1[...] = o1
                ox1[...] = o2
                oy2[...] = o3
                ox2[...] = o4
                return nact

            lax.while_loop(lambda c: c > 0, body, jnp.int32(1))


def _decode_nms(planes):
    outs = pl.pallas_call(
        _nms_body,
        out_shape=[jax.ShapeDtypeStruct((_B, 1024), jnp.float32)] * 4,
        scratch_shapes=[pltpu.VMEM((_B, _TOPK), jnp.float32)] * 5
        + [pltpu.VMEM((_B, _W), jnp.int32),
           pltpu.VMEM((_B, 128), jnp.int32)],
    )(*planes)
    return outs


def kernel(class_probs, bbox_offset, anchors):
    scores = class_probs[:, :, 1]
    spad = jnp.pad(scores, ((0, 0), (0, _NSORT - _N)),
                   constant_values=-jnp.inf).reshape(_B, _ROWS, 128)
    idx3 = _topk_indices(spad)
    idx2 = idx3.reshape(_NTILES, _CHUNKS, 128)
    big = jnp.concatenate([anchors.reshape(_B * _N, 4),
                           bbox_offset.reshape(_B * _N, 4)], axis=1)
    wide = jnp.pad(big, ((0, 0), (0, 120)))
    g = _sc_gather(idx2, wide)
    planes = tuple(g[:, c].reshape(_B, _TOPK) for c in range(8))
    oy1, ox1, oy2, ox2 = _decode_nms(planes)
    props = jnp.stack([oy1, ox1, oy2, ox2], axis=-1)[:, :_MAXOUT, :]
    return props


# speedup vs baseline: 1.3102x; 1.0475x over previous
import functools

import jax
import jax.numpy as jnp
import numpy as np
from jax import lax
from jax.experimental import pallas as pl
from jax.experimental.pallas import tpu as pltpu
from jax.experimental.pallas import tpu_sc as plsc

_B = 8
_N = 20000
_PRE = 6000
_MAXOUT = 1000
_THR = 0.7
_NSORT = 32768
_ROWS = 256
_TOPROWS = 48
_TOPK = _TOPROWS * 128


def _stage_params():
    rows = []
    kk = 2
    while kk <= _NSORT:
        j = kk // 2
        while j >= 1:
            dr = j // 128 if j >= 128 else 0
            jl = j if j < 128 else 0
            rows.append((j, kk, dr, jl))
            j //= 2
        kk *= 2
    return np.asarray(rows, dtype=np.int32)


_PARAMS = _stage_params()
_NSTAGES = _PARAMS.shape[0]


def _sort_body(params_ref, scores_ref, out_ref, key_ref, idx_ref, lin_ref):
    pid = pl.program_id(0)

    @pl.when(pid == 0)
    def _init():
        s = scores_ref[...]
        b = lax.bitcast_convert_type(s, jnp.int32)
        key = b ^ ((b >> 31) & jnp.int32(0x7FFFFFFF))
        r = lax.broadcasted_iota(jnp.int32, (_B, _ROWS, 128), 1)
        c = lax.broadcasted_iota(jnp.int32, (_B, _ROWS, 128), 2)
        lin = r * 128 + c
        lin_ref[...] = lin
        key_ref[:, 0:_ROWS, :] = key
        key_ref[:, _ROWS:, :] = key
        idx_ref[:, 0:_ROWS, :] = lin
        idx_ref[:, _ROWS:, :] = lin

    j = params_ref[pid, 0]
    kk = params_ref[pid, 1]
    dr = params_ref[pid, 2]
    jl = params_ref[pid, 3]
    sh_m = lax.rem(jnp.int32(128) - jl, jnp.int32(128))

    lin = lin_ref[...]
    xk = key_ref[:, 0:_ROWS, :]
    xi = idx_ref[:, 0:_ROWS, :]
    km = pltpu.roll(key_ref[:, pl.ds(dr, _ROWS), :], sh_m, axis=2)
    kp = pltpu.roll(key_ref[:, pl.ds(_ROWS - dr, _ROWS), :], jl, axis=2)
    im = pltpu.roll(idx_ref[:, pl.ds(dr, _ROWS), :], sh_m, axis=2)
    ip = pltpu.roll(idx_ref[:, pl.ds(_ROWS - dr, _ROWS), :], jl, axis=2)

    is_hi = (lin & j) != 0
    is_asc = (lin & kk) != 0
    pk = jnp.where(is_hi, kp, km)
    pi = jnp.where(is_hi, ip, im)
    self_wins = (xk > pk) | ((xk == pk) & (xi < pi))
    keep = self_wins ^ is_hi ^ is_asc
    nk = jnp.where(keep, xk, pk)
    ni = jnp.where(keep, xi, pi)
    key_ref[:, 0:_ROWS, :] = nk
    key_ref[:, _ROWS:, :] = nk
    idx_ref[:, 0:_ROWS, :] = ni
    idx_ref[:, _ROWS:, :] = ni

    @pl.when(pid == _NSTAGES - 1)
    def _fin():
        ii = idx_ref[:, 0:_TOPROWS, :]
        bidx = lax.broadcasted_iota(jnp.int32, (_B, _TOPROWS, 128), 0)
        out_ref[...] = ii + bidx * _N


def _topk_indices(scores_pad):
    return pl.pallas_call(
        _sort_body,
        grid_spec=pltpu.PrefetchScalarGridSpec(
            num_scalar_prefetch=1,
            grid=(_NSTAGES,),
            in_specs=[
                pl.BlockSpec((_B, _ROWS, 128), lambda i, p: (0, 0, 0)),
            ],
            out_specs=pl.BlockSpec((_B, _TOPROWS, 128), lambda i, p: (0, 0, 0)),
            scratch_shapes=[
                pltpu.VMEM((_B, 2 * _ROWS, 128), jnp.int32),
                pltpu.VMEM((_B, 2 * _ROWS, 128), jnp.int32),
                pltpu.VMEM((_B, _ROWS, 128), jnp.int32),
            ],
        ),
        out_shape=jax.ShapeDtypeStruct((_B, _TOPROWS, 128), jnp.int32),
    )(jnp.asarray(_PARAMS), scores_pad)


_NTILES = 32
_PER_TILE = (_B * _TOPK) // _NTILES
_CHUNKS = _PER_TILE // 128


def _sc_gather(idx3d, wide_tab):
    mesh = plsc.VectorSubcoreMesh(core_axis_name="c", subcore_axis_name="s")

    @functools.partial(
        pl.kernel,
        mesh=mesh,
        out_type=jax.ShapeDtypeStruct((_B * _TOPK, 128), jnp.float32),
        scratch_types=[
            pltpu.VMEM((_CHUNKS, 128), jnp.int32),
            pltpu.VMEM((128, 128), jnp.float32),
            pltpu.VMEM((128, 128), jnp.float32),
            pltpu.SemaphoreType.DMA,
            pltpu.SemaphoreType.DMA,
        ],
    )
    def gather_k(idx_hbm, tab_hbm, out, idx_v, buf0, buf1, sem0, sem1):
        wid = lax.axis_index("s") * 2 + lax.axis_index("c")
        pltpu.sync_copy(idx_hbm.at[wid], idx_v)
        bufs = (buf0, buf1)
        sems = (sem0, sem1)
        cps = [None] * _CHUNKS
        cps[0] = pltpu.async_copy(tab_hbm.at[idx_v.at[0]], bufs[0], sems[0])
        for c in range(_CHUNKS):
            if c + 1 < _CHUNKS:
                cps[c + 1] = pltpu.async_copy(tab_hbm.at[idx_v.at[c + 1]],
                                              bufs[(c + 1) % 2],
                                              sems[(c + 1) % 2])
            cps[c].wait()
            pltpu.sync_copy(bufs[c % 2],
                            out.at[pl.ds(wid * _PER_TILE + c * 128, 128), :])

    return gather_k(idx3d, wide_tab)


_BIG = 1 << 20
_W = 2048
_SPB = 2
_SEG = 256


def _nms_body(ay1, ax1, ay2, ax2, dy, dx, dh, dw,
              oy1, ox1, oy2, ox2,
              y1r, x1r, y2r, x2r, arear, remr, cntr):
    h = ay2[...] - ay1[...]
    w = ax2[...] - ax1[...]
    cy = ay1[...] + 0.5 * h
    cx = ax1[...] + 0.5 * w
    cy = cy + (dy[...] * 0.1) * h
    cx = cx + (dx[...] * 0.1) * w
    h = h * jnp.exp(dh[...] * 0.2)
    w = w * jnp.exp(dw[...] * 0.2)
    y1 = cy - 0.5 * h
    x1 = cx - 0.5 * w
    y2 = y1 + h
    x2 = x1 + w
    y1 = jnp.clip(y1, 0.0, 1.0)
    x1 = jnp.clip(x1, 0.0, 1.0)
    y2 = jnp.clip(y2, 0.0, 1.0)
    x2 = jnp.clip(x2, 0.0, 1.0)
    y1r[...] = y1
    x1r[...] = x1
    y2r[...] = y2
    x2r[...] = x2
    arear[...] = jnp.maximum(y2 - y1, 0.0) * jnp.maximum(x2 - x1, 0.0)

    cntr[...] = jnp.zeros((_B, 128), jnp.int32)
    oy1[...] = jnp.zeros((_B, 1024), jnp.float32)
    ox1[...] = jnp.zeros((_B, 1024), jnp.float32)
    oy2[...] = jnp.zeros((_B, 1024), jnp.float32)
    ox2[...] = jnp.zeros((_B, 1024), jnp.float32)
    lane_o = lax.broadcasted_iota(jnp.int32, (_B, 1024), 1)
    lin_w = lax.broadcasted_iota(jnp.int32, (_B, _W), 1)

    for w in range(_TOPK // _W):
        sl = pl.ds(w * _W, _W)
        nvalid = _PRE - w * _W
        cnt0 = cntr[...][:, 0:1]

        @pl.when(jnp.min(cnt0) < _MAXOUT)
        def _window():
            y1w = y1r[:, sl]
            x1w = x1r[:, sl]
            y2w = y2r[:, sl]
            x2w = x2r[:, sl]
            aw = arear[:, sl]
            valid = (lin_w < nvalid)

            if w == 0:
                remr[...] = valid.astype(jnp.int32)
            else:
                def ext(j, supp):
                    oh = lane_o == j
                    sy1 = jnp.max(jnp.where(oh, oy1[...], -1.0), axis=1,
                                  keepdims=True)
                    sx1 = jnp.max(jnp.where(oh, ox1[...], -1.0), axis=1,
                                  keepdims=True)
                    sy2 = jnp.max(jnp.where(oh, oy2[...], -1.0), axis=1,
                                  keepdims=True)
                    sx2 = jnp.max(jnp.where(oh, ox2[...], -1.0), axis=1,
                                  keepdims=True)
                    sarea = (jnp.maximum(sy2 - sy1, 0.0)
                             * jnp.maximum(sx2 - sx1, 0.0))
                    yy1 = jnp.maximum(sy1, y1w)
                    xx1 = jnp.maximum(sx1, x1w)
                    yy2 = jnp.minimum(sy2, y2w)
                    xx2 = jnp.minimum(sx2, x2w)
                    inter = (jnp.maximum(yy2 - yy1, 0.0)
                             * jnp.maximum(xx2 - xx1, 0.0))
                    iou = inter / (sarea + aw - inter + 1e-8)
                    return supp | (iou >= _THR).astype(jnp.int32)

                maxc = jnp.max(cntr[...][:, 0:1])
                supp0 = lax.fori_loop(0, maxc, ext,
                                      jnp.zeros((_B, _W), jnp.int32))
                remr[...] = (valid & (supp0 == 0)).astype(jnp.int32)

            lin_seg = lax.broadcasted_iota(jnp.int32, (_B, _SEG), 1)
            for seg in range(_W // _SEG):
                lo = seg * _SEG
                sfxw = _W - lo
                y1s = y1w[:, lo:lo + _SEG]
                x1s = x1w[:, lo:lo + _SEG]
                y2s = y2w[:, lo:lo + _SEG]
                x2s = x2w[:, lo:lo + _SEG]
                y1f = y1w[:, lo:]
                x1f = x1w[:, lo:]
                y2f = y2w[:, lo:]
                x2f = x2w[:, lo:]
                af = aw[:, lo:]
                lin_f = lax.broadcasted_iota(jnp.int32, (_B, sfxw), 1)

                def sbody(carry, lo=lo, sfxw=sfxw, lin_f=lin_f,
                          y1s=y1s, x1s=x1s, y2s=y2s, x2s=x2s,
                          y1f=y1f, x1f=x1f, y2f=y2f, x2f=x2f, af=af):
                    rems = remr[:, pl.ds(lo, sfxw)]
                    cnt1 = cntr[...][:, 0:1]
                    o1 = oy1[...]
                    o2 = ox1[...]
                    o3 = oy2[...]
                    o4 = ox2[...]
                    nact = jnp.int32(0)
                    for _ in range(_SPB):
                        rs = rems[:, 0:_SEG]
                        cand = jnp.min(jnp.where(rs > 0, lin_seg, _BIG),
                                       axis=1, keepdims=True)
                        cand = jnp.where(cnt1 < _MAXOUT, cand, _BIG)
                        active = cand < _BIG
                        pick_s = lin_seg == cand
                        by1 = jnp.max(jnp.where(pick_s, y1s, -1.0), axis=1,
                                      keepdims=True)
                        bx1 = jnp.max(jnp.where(pick_s, x1s, -1.0), axis=1,
                                      keepdims=True)
                        by2 = jnp.max(jnp.where(pick_s, y2s, -1.0), axis=1,
                                      keepdims=True)
                        bx2 = jnp.max(jnp.where(pick_s, x2s, -1.0), axis=1,
                                      keepdims=True)
                        barea = (jnp.maximum(by2 - by1, 0.0)
                                 * jnp.maximum(bx2 - bx1, 0.0))
                        yy1 = jnp.maximum(by1, y1f)
                        xx1 = jnp.maximum(bx1, x1f)
                        yy2 = jnp.minimum(by2, y2f)
                        xx2 = jnp.minimum(bx2, x2f)
                        inter = (jnp.maximum(yy2 - yy1, 0.0)
                                 * jnp.maximum(xx2 - xx1, 0.0))
                        iou = inter / (barea + af - inter + 1e-8)
                        supp = iou >= _THR
                        pick_f = lin_f == cand
                        rems = jnp.where(supp | pick_f, 0, rems)
                        wsel = jnp.where(active, cnt1, 2000)
                        osel = lane_o == wsel
                        o1 = jnp.where(osel,
                                       jnp.broadcast_to(by1, (_B, 1024)), o1)
                        o2 = jnp.where(osel,
                                       jnp.broadcast_to(bx1, (_B, 1024)), o2)
                        o3 = jnp.where(osel,
                                       jnp.broadcast_to(by2, (_B, 1024)), o3)
                        o4 = jnp.where(osel,
                                       jnp.broadcast_to(bx2, (_B, 1024)), o4)
                        cnt1 = cnt1 + active.astype(jnp.int32)
                        nact = jnp.sum(active.astype(jnp.int32))
                    remr[:, pl.ds(lo, sfxw)] = rems
                    cntr[...] = jnp.broadcast_to(cnt1, (_B, 128))
                    oy1[...] = o1
                    ox1[...] = o2
                    oy2[...] = o3
                    ox2[...] = o4
                    return nact

                lax.while_loop(lambda c: c > 0, sbody, jnp.int32(1))


def _decode_nms(planes):
    outs = pl.pallas_call(
        _nms_body,
        out_shape=[jax.ShapeDtypeStruct((_B, 1024), jnp.float32)] * 4,
        scratch_shapes=[pltpu.VMEM((_B, _TOPK), jnp.float32)] * 5
        + [pltpu.VMEM((_B, _W), jnp.int32),
           pltpu.VMEM((_B, 128), jnp.int32)],
    )(*planes)
    return outs


def kernel(class_probs, bbox_offset, anchors):
    scores = class_probs[:, :, 1]
    spad = jnp.pad(scores, ((0, 0), (0, _NSORT - _N)),
                   constant_values=-jnp.inf).reshape(_B, _ROWS, 128)
    idx3 = _topk_indices(spad)
    idx2 = idx3.reshape(_NTILES, _CHUNKS, 128)
    big = jnp.concatenate([anchors.reshape(_B * _N, 4),
                           bbox_offset.reshape(_B * _N, 4)], axis=1)
    wide = jnp.pad(big, ((0, 0), (0, 120)))
    g = _sc_gather(idx2, wide)
    planes = tuple(g[:, c].reshape(_B, _TOPK) for c in range(8))
    oy1, ox1, oy2, ox2 = _decode_nms(planes)
    props = jnp.stack([oy1, ox1, oy2, ox2], axis=-1)[:, :_MAXOUT, :]
    return props
